# baseline (speedup 1.0000x reference)
import jax
import jax.numpy as jnp
from jax import lax
from jax.experimental import pallas as pl
from jax.experimental.pallas import tpu as pltpu
from jax.experimental.pallas import tpu_sc as plsc

_B, _T, _V = 32, 9, 8192
_N = _B * _T
_NC, _NS = 2, 16
_NW = _NC * _NS
_RPC = 8
_CW = 1024
_NCOL = _V // _CW
_NUNIT = (_N // _RPC) * _NCOL
_MAXU = -(-_NUNIT // _NW)


def _sc_gather_body(table_hbm, xf_hbm, out_hbm, *sc):
    idx_all = sc[0]
    buf = sc[1:1 + _MAXU]
    semg = sc[1 + _MAXU:1 + 2 * _MAXU]
    semw = sc[1 + 2 * _MAXU:1 + 3 * _MAXU]
    wid = lax.axis_index("s") * _NC + lax.axis_index("c")
    pltpu.sync_copy(xf_hbm, idx_all)

    def unit(k):
        u = wid + _NW * k
        c = u // _NCOL
        b = u % _NCOL
        row = pl.multiple_of(c * _RPC, _RPC)
        col = pl.multiple_of(b * _CW, _CW)
        return row, col

    def fire(k):
        row, col = unit(k)
        pltpu.async_copy(
            table_hbm.at[idx_all.at[pl.ds(row, _RPC)], pl.ds(col, _CW)],
            buf[k], semg[k])

    def write(k):
        row, col = unit(k)
        pltpu.make_async_copy(
            table_hbm.at[idx_all.at[pl.ds(row, _RPC)], pl.ds(col, _CW)],
            buf[k], semg[k]).wait()
        pltpu.async_copy(
            buf[k], out_hbm.at[pl.ds(row, _RPC), pl.ds(col, _CW)], semw[k])

    def drain(k):
        row, col = unit(k)
        pltpu.make_async_copy(
            buf[k], out_hbm.at[pl.ds(row, _RPC), pl.ds(col, _CW)],
            semw[k]).wait()

    nfull = _NUNIT // _NW
    extra = _NUNIT - nfull * _NW
    has_extra = wid < extra
    for k in range(nfull):
        fire(k)
    if extra:
        pl.when(has_extra)(lambda: fire(nfull))
    for k in range(nfull):
        write(k)
    if extra:
        pl.when(has_extra)(lambda: write(nfull))
    for k in range(nfull):
        drain(k)
    if extra:
        pl.when(has_extra)(lambda: drain(nfull))


def _sc_gather(table, xf):
    mesh = plsc.VectorSubcoreMesh(core_axis_name="c", subcore_axis_name="s")
    f = pl.kernel(
        _sc_gather_body,
        out_type=jax.ShapeDtypeStruct((_N, _V), jnp.float32),
        mesh=mesh,
        scratch_types=(
            [pltpu.VMEM((_N,), jnp.int32)]
            + [pltpu.VMEM((_RPC, _CW), jnp.float32)] * _MAXU
            + [pltpu.SemaphoreType.DMA] * (2 * _MAXU)
        ),
    )
    return f(table, xf)


_RG = 8
_GL = _N // _RG


def _tc_gloss_body(xf_sm, table_ref, y_ref, loss_ref, rows2, sems, acc_ref):
    i = pl.program_id(0)
    slot = lax.rem(i, 2)
    nslot = lax.rem(i + 1, 2)

    def issue(step, sl):
        for j in range(_RG):
            r = xf_sm[step * _RG + j]
            pltpu.make_async_copy(
                table_ref.at[pl.ds(r, 1), :],
                rows2.at[sl, pl.ds(j, 1), :],
                sems.at[sl],
            ).start()

    @pl.when(i == 0)
    def _():
        acc_ref[0] = 0.0
        issue(i, slot)

    @pl.when(i + 1 < _GL)
    def _():
        issue(i + 1, nslot)

    pltpu.make_async_copy(
        table_ref.at[pl.ds(0, _RG), :], rows2.at[slot], sems.at[slot]
    ).wait()

    lg = rows2.at[slot][...]
    m = jnp.max(lg, axis=1)
    s = jnp.sum(jnp.exp(lg - m[:, None]), axis=1)
    ids = lax.broadcasted_iota(jnp.int32, (_RG, _V), 1)
    t = jnp.sum(jnp.where(ids == y_ref[...], lg, 0.0), axis=1)
    acc_ref[0] += jnp.sum(t - m - jnp.log(s))

    @pl.when(i == _GL - 1)
    def _():
        loss_ref[0, 0] = -acc_ref[0] / _N


def _tc_gloss(table, xf, y2):
    grid_spec = pltpu.PrefetchScalarGridSpec(
        num_scalar_prefetch=1,
        grid=(_GL,),
        in_specs=[
            pl.BlockSpec(memory_space=pl.ANY),
            pl.BlockSpec((_RG, 1), lambda i, xf_sm: (i, 0)),
        ],
        out_specs=pl.BlockSpec(memory_space=pltpu.SMEM),
        scratch_shapes=[
            pltpu.VMEM((2, _RG, _V), jnp.float32),
            pltpu.SemaphoreType.DMA((2,)),
            pltpu.SMEM((1,), jnp.float32),
        ],
    )
    return pl.pallas_call(
        _tc_gloss_body,
        grid_spec=grid_spec,
        out_shape=jax.ShapeDtypeStruct((1, 1), jnp.float32),
    )(xf, table, y2)


def kernel(x, y, embedding_table):
    xf = x.reshape(_N).astype(jnp.int32)
    logits = _sc_gather(embedding_table, xf)
    y2 = y.reshape(_N, 1).astype(jnp.int32)
    loss = _tc_gloss(embedding_table, xf, y2)
    return (logits, loss.reshape(()))


# speedup vs baseline: 1.4148x; 1.4148x over previous
import jax
import jax.numpy as jnp
from jax import lax
from jax.experimental import pallas as pl
from jax.experimental.pallas import tpu as pltpu
from jax.experimental.pallas import tpu_sc as plsc

_B, _T, _V = 32, 9, 8192
_N = _B * _T
_NC, _NS = 2, 16
_NW = _NC * _NS
_RPC = 8
_CW = 2048
_NCOL = _V // _CW
_NUNIT = (_N // _RPC) * _NCOL
_MAXU = -(-_NUNIT // _NW)


def _sc_gather_body(table_hbm, xf_hbm, out_hbm, *sc):
    idx_all = sc[0]
    buf = sc[1:1 + _MAXU]
    semg = sc[1 + _MAXU:1 + 2 * _MAXU]
    semw = sc[1 + 2 * _MAXU:1 + 3 * _MAXU]
    wid = lax.axis_index("s") * _NC + lax.axis_index("c")
    pltpu.sync_copy(xf_hbm, idx_all)

    def unit(k):
        u = wid + _NW * k
        c = u // _NCOL
        b = u % _NCOL
        row = pl.multiple_of(c * _RPC, _RPC)
        col = pl.multiple_of(b * _CW, _CW)
        return row, col

    def fire(k):
        row, col = unit(k)
        pltpu.async_copy(
            table_hbm.at[idx_all.at[pl.ds(row, _RPC)], pl.ds(col, _CW)],
            buf[k], semg[k])

    def write(k):
        row, col = unit(k)
        pltpu.make_async_copy(
            table_hbm.at[idx_all.at[pl.ds(row, _RPC)], pl.ds(col, _CW)],
            buf[k], semg[k]).wait()
        pltpu.async_copy(
            buf[k], out_hbm.at[pl.ds(row, _RPC), pl.ds(col, _CW)], semw[k])

    def drain(k):
        row, col = unit(k)
        pltpu.make_async_copy(
            buf[k], out_hbm.at[pl.ds(row, _RPC), pl.ds(col, _CW)],
            semw[k]).wait()

    nfull = _NUNIT // _NW
    extra = _NUNIT - nfull * _NW
    has_extra = wid < extra
    for k in range(nfull):
        fire(k)
    if extra:
        pl.when(has_extra)(lambda: fire(nfull))
    for k in range(nfull):
        write(k)
    if extra:
        pl.when(has_extra)(lambda: write(nfull))
    for k in range(nfull):
        drain(k)
    if extra:
        pl.when(has_extra)(lambda: drain(nfull))


def _sc_gather(table, xf):
    mesh = plsc.VectorSubcoreMesh(core_axis_name="c", subcore_axis_name="s")
    f = pl.kernel(
        _sc_gather_body,
        out_type=jax.ShapeDtypeStruct((_N, _V), jnp.float32),
        mesh=mesh,
        scratch_types=(
            [pltpu.VMEM((_N,), jnp.int32)]
            + [pltpu.VMEM((_RPC, _CW), jnp.float32)] * _MAXU
            + [pltpu.SemaphoreType.DMA] * (2 * _MAXU)
        ),
    )
    return f(table, xf)


_ROWS_PER_BLK = 96
_NBLK = _N // _ROWS_PER_BLK


def _tc_loss_body(lg_ref, y_ref, loss_ref, acc_ref):
    i = pl.program_id(0)
    lg = lg_ref[...]
    m = jnp.max(lg, axis=1)
    s = jnp.sum(jnp.exp(lg - m[:, None]), axis=1)
    ids = lax.broadcasted_iota(jnp.int32, (_ROWS_PER_BLK, _V), 1)
    t = jnp.sum(jnp.where(ids == y_ref[...], lg, 0.0), axis=1)
    part = jnp.sum(t - m - jnp.log(s))

    @pl.when(i == 0)
    def _():
        acc_ref[0] = 0.0

    acc_ref[0] += part

    @pl.when(i == _NBLK - 1)
    def _():
        loss_ref[0, 0] = -acc_ref[0] / _N


def _tc_loss(logits, y2):
    return pl.pallas_call(
        _tc_loss_body,
        grid=(_NBLK,),
        in_specs=[
            pl.BlockSpec((_ROWS_PER_BLK, _V), lambda i: (i, 0)),
            pl.BlockSpec((_ROWS_PER_BLK, 1), lambda i: (i, 0)),
        ],
        out_specs=pl.BlockSpec(memory_space=pltpu.SMEM),
        out_shape=jax.ShapeDtypeStruct((1, 1), jnp.float32),
        scratch_shapes=[pltpu.SMEM((1,), jnp.float32)],
    )(logits, y2)


def kernel(x, y, embedding_table):
    xf = x.reshape(_N).astype(jnp.int32)
    logits = _sc_gather(embedding_table, xf)
    y2 = y.reshape(_N, 1).astype(jnp.int32)
    loss = _tc_loss(logits, y2)
    return (logits, loss.reshape(()))


# speedup vs baseline: 1.4519x; 1.0262x over previous
import jax
import jax.numpy as jnp
from jax import lax
from jax.experimental import pallas as pl
from jax.experimental.pallas import tpu as pltpu
from jax.experimental.pallas import tpu_sc as plsc

_B, _T, _V = 32, 9, 8192
_N = _B * _T
_NC, _NS = 2, 16
_NW = _NC * _NS
_RPC = 8
_CW = 1024
_NCOL = _V // _CW
_NUNIT = (_N // _RPC) * _NCOL
_MAXU = -(-_NUNIT // _NW)


def _sc_gather_body(table_hbm, xf_hbm, out_hbm, *sc):
    idx_all = sc[0]
    buf = sc[1:1 + _MAXU]
    semg = sc[1 + _MAXU:1 + 2 * _MAXU]
    semw = sc[1 + 2 * _MAXU:1 + 3 * _MAXU]
    wid = lax.axis_index("s") * _NC + lax.axis_index("c")
    pltpu.sync_copy(xf_hbm, idx_all)

    def unit(k):
        u = wid + _NW * k
        c = u // _NCOL
        b = u % _NCOL
        row = pl.multiple_of(c * _RPC, _RPC)
        col = pl.multiple_of(b * _CW, _CW)
        return row, col

    def fire(k):
        row, col = unit(k)
        pltpu.async_copy(
            table_hbm.at[idx_all.at[pl.ds(row, _RPC)], pl.ds(col, _CW)],
            buf[k], semg[k])

    def write(k):
        row, col = unit(k)
        pltpu.make_async_copy(
            table_hbm.at[idx_all.at[pl.ds(row, _RPC)], pl.ds(col, _CW)],
            buf[k], semg[k]).wait()
        pltpu.async_copy(
            buf[k], out_hbm.at[pl.ds(row, _RPC), pl.ds(col, _CW)], semw[k])

    def drain(k):
        row, col = unit(k)
        pltpu.make_async_copy(
            buf[k], out_hbm.at[pl.ds(row, _RPC), pl.ds(col, _CW)],
            semw[k]).wait()

    nfull = _NUNIT // _NW
    extra = _NUNIT - nfull * _NW
    has_extra = wid < extra
    for k in range(nfull):
        fire(k)
    if extra:
        pl.when(has_extra)(lambda: fire(nfull))
    for k in range(nfull):
        write(k)
    if extra:
        pl.when(has_extra)(lambda: write(nfull))
    for k in range(nfull):
        drain(k)
    if extra:
        pl.when(has_extra)(lambda: drain(nfull))


def _sc_gather(table, xf):
    mesh = plsc.VectorSubcoreMesh(core_axis_name="c", subcore_axis_name="s")
    f = pl.kernel(
        _sc_gather_body,
        out_type=jax.ShapeDtypeStruct((_N, _V), jnp.float32),
        mesh=mesh,
        scratch_types=(
            [pltpu.VMEM((_N,), jnp.int32)]
            + [pltpu.VMEM((_RPC, _CW), jnp.float32)] * _MAXU
            + [pltpu.SemaphoreType.DMA] * (2 * _MAXU)
        ),
    )
    return f(table, xf)


_ROWS_PER_BLK = 96
_NBLK = _N // _ROWS_PER_BLK


def _tc_loss_body(lg_ref, y_ref, loss_ref, acc_ref):
    i = pl.program_id(0)
    lg = lg_ref[...]
    m = jnp.max(lg, axis=1)
    s = jnp.sum(jnp.exp(lg - m[:, None]), axis=1)
    ids = lax.broadcasted_iota(jnp.int32, (_ROWS_PER_BLK, _V), 1)
    t = jnp.sum(jnp.where(ids == y_ref[...], lg, 0.0), axis=1)
    part = jnp.sum(t - m - jnp.log(s))

    @pl.when(i == 0)
    def _():
        acc_ref[0] = 0.0

    acc_ref[0] += part

    @pl.when(i == _NBLK - 1)
    def _():
        loss_ref[0, 0] = -acc_ref[0] / _N


def _tc_loss(logits, y2):
    return pl.pallas_call(
        _tc_loss_body,
        grid=(_NBLK,),
        in_specs=[
            pl.BlockSpec((_ROWS_PER_BLK, _V), lambda i: (i, 0)),
            pl.BlockSpec((_ROWS_PER_BLK, 1), lambda i: (i, 0)),
        ],
        out_specs=pl.BlockSpec(memory_space=pltpu.SMEM),
        out_shape=jax.ShapeDtypeStruct((1, 1), jnp.float32),
        scratch_shapes=[pltpu.SMEM((1,), jnp.float32)],
    )(logits, y2)


def kernel(x, y, embedding_table):
    xf = x.reshape(_N).astype(jnp.int32)
    logits = _sc_gather(embedding_table, xf)
    y2 = y.reshape(_N, 1).astype(jnp.int32)
    loss = _tc_loss(logits, y2)
    return (logits, loss.reshape(()))
